# baseline (speedup 1.0000x reference)
import jax
import jax.numpy as jnp
from jax.experimental import pallas as pl
from jax.experimental.pallas import tpu as pltpu


def _scan_kernel(x_ref, tri_ref, o_ref, carry_ref):
    j = pl.program_id(1)

    @pl.when(j == 0)
    def _():
        carry_ref[...] = jnp.zeros_like(carry_ref)

    xb = x_ref[...]
    part = jnp.dot(xb, tri_ref[...], preferred_element_type=jnp.float32)
    o_ref[...] = part + carry_ref[...][:, :1]
    carry_ref[...] = carry_ref[...] + jnp.sum(xb, axis=1, keepdims=True)


def kernel(x):
    n_rows, n_cols = x.shape
    BR = 256
    BC = 128
    grid = (n_rows // BR, n_cols // BC)

    col = jax.lax.broadcasted_iota(jnp.int32, (BC, BC), 1)
    row = jax.lax.broadcasted_iota(jnp.int32, (BC, BC), 0)
    tri = (row < col).astype(jnp.float32)

    return pl.pallas_call(
        _scan_kernel,
        grid=grid,
        in_specs=[
            pl.BlockSpec((BR, BC), lambda i, j: (i, j)),
            pl.BlockSpec((BC, BC), lambda i, j: (0, 0)),
        ],
        out_specs=pl.BlockSpec((BR, BC), lambda i, j: (i, j)),
        out_shape=jax.ShapeDtypeStruct((n_rows, n_cols), jnp.float32),
        scratch_shapes=[pltpu.VMEM((BR, 128), jnp.float32)],
        compiler_params=pltpu.CompilerParams(
            dimension_semantics=("parallel", "arbitrary"),
        ),
    )(x, tri)


# speedup vs baseline: 2.8660x; 2.8660x over previous
import jax
import jax.numpy as jnp
from jax.experimental import pallas as pl
from jax.experimental.pallas import tpu as pltpu


_CHUNK = 128


def _scan_kernel(x_ref, tri_ref, o_ref, carry_ref):
    j = pl.program_id(1)

    @pl.when(j == 0)
    def _():
        carry_ref[...] = jnp.zeros_like(carry_ref)

    xb = x_ref[...]
    tri = tri_ref[...]
    bc = xb.shape[1]
    parts = []
    chunk_carry = jnp.zeros((xb.shape[0], 1), dtype=jnp.float32)
    for k in range(bc // _CHUNK):
        chunk = xb[:, k * _CHUNK:(k + 1) * _CHUNK]
        p = jnp.dot(chunk, tri, preferred_element_type=jnp.float32)
        parts.append(p + chunk_carry)
        chunk_carry = chunk_carry + jnp.sum(chunk, axis=1, keepdims=True)
    o_ref[...] = jnp.concatenate(parts, axis=1) + carry_ref[...][:, :1]
    carry_ref[...] = carry_ref[...] + chunk_carry


def kernel(x):
    n_rows, n_cols = x.shape
    BR = 256
    BC = 512
    grid = (n_rows // BR, n_cols // BC)

    col = jax.lax.broadcasted_iota(jnp.int32, (_CHUNK, _CHUNK), 1)
    row = jax.lax.broadcasted_iota(jnp.int32, (_CHUNK, _CHUNK), 0)
    tri = (row < col).astype(jnp.float32)

    return pl.pallas_call(
        _scan_kernel,
        grid=grid,
        in_specs=[
            pl.BlockSpec((BR, BC), lambda i, j: (i, j)),
            pl.BlockSpec((_CHUNK, _CHUNK), lambda i, j: (0, 0)),
        ],
        out_specs=pl.BlockSpec((BR, BC), lambda i, j: (i, j)),
        out_shape=jax.ShapeDtypeStruct((n_rows, n_cols), jnp.float32),
        scratch_shapes=[pltpu.VMEM((BR, 128), jnp.float32)],
        compiler_params=pltpu.CompilerParams(
            dimension_semantics=("parallel", "arbitrary"),
        ),
    )(x, tri)


# speedup vs baseline: 5.5108x; 1.9229x over previous
import jax
import jax.numpy as jnp
from jax.experimental import pallas as pl
from jax.experimental.pallas import tpu as pltpu


_CHUNK = 128


def _scan_kernel(x_ref, tri_ref, o_ref, carry_ref):
    j = pl.program_id(1)

    @pl.when(j == 0)
    def _():
        carry_ref[...] = jnp.zeros_like(carry_ref)

    xb = x_ref[...]
    tri = tri_ref[...]
    bc = xb.shape[1]
    parts = []
    chunk_carry = jnp.zeros((xb.shape[0], 1), dtype=jnp.float32)
    for k in range(bc // _CHUNK):
        chunk = xb[:, k * _CHUNK:(k + 1) * _CHUNK]
        p = jnp.dot(chunk, tri, preferred_element_type=jnp.float32)
        parts.append(p + chunk_carry)
        chunk_carry = chunk_carry + jnp.sum(chunk, axis=1, keepdims=True)
    o_ref[...] = jnp.concatenate(parts, axis=1) + carry_ref[...][:, :1]
    carry_ref[...] = carry_ref[...] + chunk_carry


def kernel(x):
    n_rows, n_cols = x.shape
    BR = 512
    BC = 1024
    grid = (n_rows // BR, n_cols // BC)

    col = jax.lax.broadcasted_iota(jnp.int32, (_CHUNK, _CHUNK), 1)
    row = jax.lax.broadcasted_iota(jnp.int32, (_CHUNK, _CHUNK), 0)
    tri = (row < col).astype(jnp.float32)

    return pl.pallas_call(
        _scan_kernel,
        grid=grid,
        in_specs=[
            pl.BlockSpec((BR, BC), lambda i, j: (i, j)),
            pl.BlockSpec((_CHUNK, _CHUNK), lambda i, j: (0, 0)),
        ],
        out_specs=pl.BlockSpec((BR, BC), lambda i, j: (i, j)),
        out_shape=jax.ShapeDtypeStruct((n_rows, n_cols), jnp.float32),
        scratch_shapes=[pltpu.VMEM((BR, 128), jnp.float32)],
        compiler_params=pltpu.CompilerParams(
            dimension_semantics=("parallel", "arbitrary"),
        ),
    )(x, tri)


# speedup vs baseline: 6.4372x; 1.1681x over previous
import jax
import jax.numpy as jnp
from jax.experimental import pallas as pl
from jax.experimental.pallas import tpu as pltpu


_CHUNK = 128


def _scan_kernel(x_ref, tri_ref, o_ref, carry_ref):
    j = pl.program_id(1)

    @pl.when(j == 0)
    def _():
        carry_ref[...] = jnp.zeros_like(carry_ref)

    xb = x_ref[...]
    tri = tri_ref[...]
    bc = xb.shape[1]
    parts = []
    chunk_carry = jnp.zeros((xb.shape[0], 1), dtype=jnp.float32)
    for k in range(bc // _CHUNK):
        chunk = xb[:, k * _CHUNK:(k + 1) * _CHUNK]
        p = jnp.dot(chunk, tri, preferred_element_type=jnp.float32)
        parts.append(p + chunk_carry)
        chunk_carry = chunk_carry + jnp.sum(chunk, axis=1, keepdims=True)
    o_ref[...] = jnp.concatenate(parts, axis=1) + carry_ref[...][:, :1]
    carry_ref[...] = carry_ref[...] + chunk_carry


def kernel(x):
    n_rows, n_cols = x.shape
    BR = 1024
    BC = 1024
    grid = (n_rows // BR, n_cols // BC)

    col = jax.lax.broadcasted_iota(jnp.int32, (_CHUNK, _CHUNK), 1)
    row = jax.lax.broadcasted_iota(jnp.int32, (_CHUNK, _CHUNK), 0)
    tri = (row < col).astype(jnp.float32)

    return pl.pallas_call(
        _scan_kernel,
        grid=grid,
        in_specs=[
            pl.BlockSpec((BR, BC), lambda i, j: (i, j)),
            pl.BlockSpec((_CHUNK, _CHUNK), lambda i, j: (0, 0)),
        ],
        out_specs=pl.BlockSpec((BR, BC), lambda i, j: (i, j)),
        out_shape=jax.ShapeDtypeStruct((n_rows, n_cols), jnp.float32),
        scratch_shapes=[pltpu.VMEM((BR, 128), jnp.float32)],
        compiler_params=pltpu.CompilerParams(
            dimension_semantics=("parallel", "arbitrary"),
        ),
    )(x, tri)


# speedup vs baseline: 6.9444x; 1.0788x over previous
import jax
import jax.numpy as jnp
from jax.experimental import pallas as pl
from jax.experimental.pallas import tpu as pltpu


_CHUNK = 128


def _scan_kernel(x_ref, tri_ref, o_ref, carry_ref):
    j = pl.program_id(1)

    @pl.when(j == 0)
    def _():
        carry_ref[...] = jnp.zeros_like(carry_ref)

    xb = x_ref[...]
    tri = tri_ref[...]
    bc = xb.shape[1]
    parts = []
    chunk_carry = jnp.zeros((xb.shape[0], 1), dtype=jnp.float32)
    for k in range(bc // _CHUNK):
        chunk = xb[:, k * _CHUNK:(k + 1) * _CHUNK]
        p = jnp.dot(chunk, tri, preferred_element_type=jnp.float32)
        parts.append(p + chunk_carry)
        chunk_carry = chunk_carry + jnp.sum(chunk, axis=1, keepdims=True)
    o_ref[...] = jnp.concatenate(parts, axis=1) + carry_ref[...][:, :1]
    carry_ref[...] = carry_ref[...] + chunk_carry


def kernel(x):
    n_rows, n_cols = x.shape
    BR = 1024
    BC = 2048
    grid = (n_rows // BR, n_cols // BC)

    col = jax.lax.broadcasted_iota(jnp.int32, (_CHUNK, _CHUNK), 1)
    row = jax.lax.broadcasted_iota(jnp.int32, (_CHUNK, _CHUNK), 0)
    tri = (row < col).astype(jnp.float32)

    return pl.pallas_call(
        _scan_kernel,
        grid=grid,
        in_specs=[
            pl.BlockSpec((BR, BC), lambda i, j: (i, j)),
            pl.BlockSpec((_CHUNK, _CHUNK), lambda i, j: (0, 0)),
        ],
        out_specs=pl.BlockSpec((BR, BC), lambda i, j: (i, j)),
        out_shape=jax.ShapeDtypeStruct((n_rows, n_cols), jnp.float32),
        scratch_shapes=[pltpu.VMEM((BR, 128), jnp.float32)],
        compiler_params=pltpu.CompilerParams(
            dimension_semantics=("parallel", "arbitrary"),
        ),
    )(x, tri)
